# Initial kernel scaffold; baseline (speedup 1.0000x reference)
import functools

import jax
import jax.numpy as jnp
from jax import lax
from jax.experimental import pallas as pl
from jax.experimental.pallas import tpu as pltpu
from jax.experimental.pallas import tpu_sc as plsc

NC = 2
NS = 16
NW = NC * NS

B = 4096 * 200
C = 32
N_PER_W = B // NW
CHUNK = 1024
N_CHUNKS = N_PER_W // CHUNK

_MESH = plsc.VectorSubcoreMesh(
    core_axis_name="c", subcore_axis_name="s", num_cores=NC, num_subcores=NS
)


@functools.partial(
    pl.kernel,
    out_type=jax.ShapeDtypeStruct((B, C), jnp.float32),
    mesh=_MESH,
    scratch_types=[
        pltpu.VMEM((CHUNK,), jnp.int32),
        pltpu.VMEM((CHUNK, C), jnp.float32),
        pltpu.SemaphoreType.DMA,
    ],
)
def _grid_gather(idx_hbm, table_hbm, out_hbm, idx_v, rows_v, sem):
    wid = lax.axis_index("s") * NC + lax.axis_index("c")
    base = wid * N_PER_W

    def step(i, carry):
        off = base + i * CHUNK
        pltpu.sync_copy(idx_hbm.at[pl.ds(off, CHUNK)], idx_v)
        pltpu.async_copy(table_hbm.at[idx_v], rows_v, sem).wait()
        pltpu.sync_copy(rows_v, out_hbm.at[pl.ds(off, CHUNK)])
        return carry

    lax.fori_loop(0, N_CHUNKS, step, 0)


def kernel(t, grid):
    flat_idx = t.reshape(-1).astype(jnp.int32)
    out = _grid_gather(flat_idx, grid)
    return out.reshape(t.shape + (grid.shape[1],))


# speedup vs baseline: 1.4573x; 1.4573x over previous
import functools

import jax
import jax.numpy as jnp
from jax import lax
from jax.experimental import pallas as pl
from jax.experimental.pallas import tpu as pltpu
from jax.experimental.pallas import tpu_sc as plsc

NC = 2
NS = 16
NW = NC * NS

B = 4096 * 200
C = 32
N_PER_W = B // NW
CHUNK = 1024
N_CHUNKS = N_PER_W // CHUNK

_MESH = plsc.VectorSubcoreMesh(
    core_axis_name="c", subcore_axis_name="s", num_cores=NC, num_subcores=NS
)


@functools.partial(
    pl.kernel,
    out_type=jax.ShapeDtypeStruct((B, C), jnp.float32),
    mesh=_MESH,
    scratch_types=[
        pltpu.VMEM((CHUNK,), jnp.int32),
        pltpu.VMEM((CHUNK, C), jnp.float32),
        pltpu.SemaphoreType.DMA,
    ],
    compiler_params=pltpu.CompilerParams(use_tc_tiling_on_sc=False),
)
def _grid_gather(idx_hbm, table_hbm, out_hbm, idx_v, rows_v, sem):
    wid = lax.axis_index("s") * NC + lax.axis_index("c")
    base = wid * N_PER_W

    def step(i, carry):
        off = base + i * CHUNK
        pltpu.sync_copy(idx_hbm.at[pl.ds(off, CHUNK)], idx_v)
        pltpu.async_copy(table_hbm.at[idx_v], rows_v, sem).wait()
        pltpu.sync_copy(rows_v, out_hbm.at[pl.ds(off, CHUNK)])
        return carry

    lax.fori_loop(0, N_CHUNKS, step, 0)


def kernel(t, grid):
    flat_idx = t.reshape(-1).astype(jnp.int32)
    out = _grid_gather(flat_idx, grid)
    return out.reshape(t.shape + (grid.shape[1],))


# speedup vs baseline: 1.5022x; 1.0308x over previous
import functools

import jax
import jax.numpy as jnp
from jax import lax
from jax.experimental import pallas as pl
from jax.experimental.pallas import tpu as pltpu
from jax.experimental.pallas import tpu_sc as plsc

NC = 2
NS = 16
NW = NC * NS

B = 4096 * 200
C = 32
N_PER_W = B // NW
CHUNK = 800
NBUF = 4
N_CHUNKS = N_PER_W // CHUNK
N_OUTER = N_CHUNKS // NBUF

_MESH = plsc.VectorSubcoreMesh(
    core_axis_name="c", subcore_axis_name="s", num_cores=NC, num_subcores=NS
)


@functools.partial(
    pl.kernel,
    out_type=jax.ShapeDtypeStruct((B, C), jnp.float32),
    mesh=_MESH,
    scratch_types=[
        pltpu.VMEM((N_PER_W,), jnp.int32),
        [pltpu.VMEM((CHUNK, C), jnp.float32) for _ in range(NBUF)],
        [pltpu.SemaphoreType.DMA for _ in range(NBUF)],
    ],
    compiler_params=pltpu.CompilerParams(use_tc_tiling_on_sc=False),
)
def _grid_gather(idx_hbm, table_hbm, out_hbm, idx_v, rows, sems):
    wid = lax.axis_index("s") * NC + lax.axis_index("c")
    base = wid * N_PER_W

    pltpu.sync_copy(idx_hbm.at[pl.ds(base, N_PER_W)], idx_v)

    def fire(chunk, b):
        pltpu.async_copy(
            table_hbm.at[idx_v.at[pl.ds(chunk * CHUNK, CHUNK)]], rows[b], sems[b]
        )

    for b in range(NBUF):
        fire(b, b)

    def outer(g, carry):
        first = g * NBUF
        for b in range(NBUF):
            pltpu.make_async_copy(
                table_hbm.at[idx_v.at[pl.ds(0, CHUNK)]], rows[b], sems[b]
            ).wait()
            pltpu.sync_copy(rows[b], out_hbm.at[pl.ds(base + (first + b) * CHUNK, CHUNK)])
            nxt = first + b + NBUF

            @pl.when(nxt < N_CHUNKS)
            def _():
                fire(nxt, b)

        return carry

    lax.fori_loop(0, N_OUTER, outer, 0)


def kernel(t, grid):
    flat_idx = t.reshape(-1).astype(jnp.int32)
    out = _grid_gather(flat_idx, grid)
    return out.reshape(t.shape + (grid.shape[1],))
